# baseline (speedup 1.0000x reference)
import functools

import jax
import jax.numpy as jnp
from jax import lax
from jax.experimental import pallas as pl
from jax.experimental.pallas import tpu as pltpu
from jax.experimental.pallas import tpu_sc as plsc

_NC = 2
_NS = 16
_NW = _NC * _NS
_LANES = 16
_CH = 128
_GRP = 8


def _make_sc_gather(F, V, D, per_w, n_rows, n_grp):
    mesh = plsc.VectorSubcoreMesh(
        core_axis_name="c", subcore_axis_name="s",
        num_cores=_NC, num_subcores=_NS)
    grp_rows = _CH * _GRP

    @functools.partial(
        pl.kernel,
        out_type=jax.ShapeDtypeStruct((_NW, n_grp, grp_rows, D), jnp.float32),
        mesh=mesh,
        scratch_types=[
            pltpu.VMEM((n_rows, _CH), jnp.int32),
            pltpu.VMEM((grp_rows, D), jnp.float32),
            pltpu.VMEM((grp_rows, D), jnp.float32),
            pltpu.SemaphoreType.DMA,
            pltpu.SemaphoreType.DMA,
            pltpu.SemaphoreType.DMA,
            pltpu.SemaphoreType.DMA,
        ],
        compiler_params=pltpu.CompilerParams(use_tc_tiling_on_sc=False),
    )
    def sc_gather(tbl_hbm, x_hbm, out_hbm, idx_v, s0, s1, g0, g1, o0, o1):
        wid = lax.axis_index("s") * _NC + lax.axis_index("c")
        base = wid * per_w

        pltpu.sync_copy(x_hbm.at[wid], idx_v)

        def prep(i, carry):
            row0 = base + i * _CH
            for k in range(_CH // _LANES):
                pos = row0 + k * _LANES + lax.iota(jnp.int32, _LANES)
                fld = lax.rem(pos, F)
                raw = idx_v[i, pl.ds(k * _LANES, _LANES)]
                val = jnp.minimum(jnp.maximum(raw, 0), V - 1) + fld * V
                idx_v[i, pl.ds(k * _LANES, _LANES)] = val
            return carry

        lax.fori_loop(0, n_rows, prep, 0)

        stages = (s0, s1)
        gsems = (g0, g1)
        osems = (o0, o1)

        def fire(g, stage, sem):
            hs = []
            for j in range(_GRP):
                hs.append(pltpu.async_copy(
                    tbl_hbm.at[idx_v.at[g * _GRP + j]],
                    stage.at[pl.ds(j * _CH, _CH)], sem))
            return hs

        pending_out = [None, None]
        hs = fire(0, stages[0], gsems[0])
        for g in range(n_grp):
            cur = g & 1
            nxt = cur ^ 1
            if g + 1 < n_grp:
                if pending_out[nxt] is not None:
                    pending_out[nxt].wait()
                    pending_out[nxt] = None
                hs_next = fire(g + 1, stages[nxt], gsems[nxt])
            for h in hs:
                h.wait()
            pending_out[cur] = pltpu.async_copy(
                stages[cur], out_hbm.at[wid, g], osems[cur])
            if g + 1 < n_grp:
                hs = hs_next
        for p in pending_out:
            if p is not None:
                p.wait()

    return sc_gather


def _tc_body(emb_ref, x_ref, wlin_ref, w1_ref, b1_ref, w2_ref, b2_ref,
             w3_ref, s_ref, c0_ref, out_ref):
    e = emb_ref[...]
    h = jnp.maximum(
        jnp.dot(e, w1_ref[...], preferred_element_type=jnp.float32)
        + b1_ref[...], 0.0)
    h = jnp.maximum(
        jnp.dot(h, w2_ref[...], preferred_element_type=jnp.float32)
        + b2_ref[...], 0.0)
    dnn = jnp.sum(h * w3_ref[...], axis=1, keepdims=True)
    se = jnp.dot(e, s_ref[...], preferred_element_type=jnp.float32)
    fm = 0.5 * (jnp.sum(se * se, axis=1, keepdims=True)
                - jnp.sum(e * e, axis=1, keepdims=True))
    xb = x_ref[...].astype(jnp.float32).astype(jnp.bfloat16).astype(jnp.float32)
    wb = wlin_ref[...].astype(jnp.bfloat16).astype(jnp.float32)
    lin = jnp.sum(xb * wb, axis=1, keepdims=True)
    z = lin + fm + dnn + c0_ref[...]
    out_ref[...] = 1.0 / (1.0 + jnp.exp(-z))


def kernel(x, tables, W_lin, b_lin, W1, b1, W2, b2, W3, b3):
    B, F = x.shape
    _, V, D = tables.shape
    N = B * F
    assert N % _NW == 0
    per_w = N // _NW
    assert per_w % (_CH * _GRP) == 0
    n_rows = per_w // _CH
    n_grp = per_w // (_CH * _GRP)

    tbl_flat = tables.reshape(F * V, D)
    x3 = x.reshape(_NW, n_rows, _CH)

    emb4 = _make_sc_gather(F, V, D, per_w, n_rows, n_grp)(tbl_flat, x3)
    emb = emb4.reshape(B, F * D)

    S = (lax.rem(lax.iota(jnp.int32, F * D), D)[:, None]
         == lax.iota(jnp.int32, D)[None, :]).astype(jnp.float32)

    Bb = 512
    grid = (B // Bb,)
    out2 = pl.pallas_call(
        _tc_body,
        grid=grid,
        in_specs=[
            pl.BlockSpec((Bb, F * D), lambda i: (i, 0)),
            pl.BlockSpec((Bb, F), lambda i: (i, 0)),
            pl.BlockSpec((1, F), lambda i: (0, 0)),
            pl.BlockSpec((F * D, 256), lambda i: (0, 0)),
            pl.BlockSpec((1, 256), lambda i: (0, 0)),
            pl.BlockSpec((256, 128), lambda i: (0, 0)),
            pl.BlockSpec((1, 128), lambda i: (0, 0)),
            pl.BlockSpec((1, 128), lambda i: (0, 0)),
            pl.BlockSpec((F * D, D), lambda i: (0, 0)),
            pl.BlockSpec((1, 1), lambda i: (0, 0)),
        ],
        out_specs=pl.BlockSpec((Bb, 1), lambda i: (i, 0)),
        out_shape=jax.ShapeDtypeStruct((B, 1), jnp.float32),
    )(
        emb, x, W_lin.reshape(1, F), W1, b1.reshape(1, 256),
        W2, b2.reshape(1, 128), W3.reshape(1, 128), S,
        (b_lin + b3).reshape(1, 1),
    )
    return out2[:, 0]
